# baseline (speedup 1.0000x reference)
import functools

import jax
import jax.numpy as jnp
from jax import lax
from jax.experimental import pallas as pl
from jax.experimental.pallas import tpu as pltpu
from jax.experimental.pallas import tpu_sc as plsc

B = 16384
D = 32
DI = 14
NC = 2
NS = 16
NW = NC * NS
BPW = B // NW
IDX_CHUNK = 128
NCHUNK = BPW // IDX_CHUNK


def _sc_gather(uidx_hbm, iidx_hbm, utab_hbm, itab_hbm, u_out, i_out,
               uidx_v, iidx_v, urows_v, irows_v, sem):
    wid = lax.axis_index("s") * NC + lax.axis_index("c")
    base = wid * BPW
    rbase = wid * NCHUNK
    pltpu.sync_copy(uidx_hbm.at[pl.ds(rbase, NCHUNK)], uidx_v)
    pltpu.sync_copy(iidx_hbm.at[pl.ds(rbase, NCHUNK)], iidx_v)
    copies = []
    for c in range(NCHUNK):
        copies.append(pltpu.async_copy(
            utab_hbm.at[uidx_v.at[c]],
            urows_v.at[pl.ds(c * IDX_CHUNK, IDX_CHUNK)], sem))
        copies.append(pltpu.async_copy(
            itab_hbm.at[iidx_v.at[c]],
            irows_v.at[pl.ds(c * IDX_CHUNK, IDX_CHUNK)], sem))
    for cp in copies:
        cp.wait()
    pltpu.sync_copy(urows_v, u_out.at[pl.ds(base, BPW)])
    pltpu.sync_copy(irows_v, i_out.at[pl.ds(base, BPW)])


def _mlp_body(u_ref, it_ref, w1_ref, b1_ref, w2_ref, b2_ref, y_ref):
    u = u_ref[:]
    it = it_ref[:]
    w1 = w1_ref[:]
    elem = u[:, :DI] * it
    h = jnp.dot(elem, w1[0:DI, :], preferred_element_type=jnp.float32)
    h = h + jnp.dot(u, w1[D:2 * D, :], preferred_element_type=jnp.float32)
    h = h + jnp.dot(it, w1[2 * D:2 * D + DI, :],
                    preferred_element_type=jnp.float32)
    h = jnp.maximum(h + b1_ref[:], 0.0)
    z = jnp.dot(h, w2_ref[:], preferred_element_type=jnp.float32) + b2_ref[:]
    y_ref[:] = jax.nn.sigmoid(z)


def kernel(group_inputs, user_inputs, item_inputs, user_table, item_table,
           W1, b1, W2, b2):
    del group_inputs
    mesh = plsc.VectorSubcoreMesh(core_axis_name="c", subcore_axis_name="s")
    gather = functools.partial(
        pl.kernel,
        out_type=[
            jax.ShapeDtypeStruct((B, D), jnp.float32),
            jax.ShapeDtypeStruct((B, DI), jnp.float32),
        ],
        mesh=mesh,
        compiler_params=pltpu.CompilerParams(use_tc_tiling_on_sc=False),
        scratch_types=[
            pltpu.VMEM((NCHUNK, IDX_CHUNK), jnp.int32),
            pltpu.VMEM((NCHUNK, IDX_CHUNK), jnp.int32),
            pltpu.VMEM((BPW, D), jnp.float32),
            pltpu.VMEM((BPW, DI), jnp.float32),
            pltpu.SemaphoreType.DMA,
        ],
    )(_sc_gather)
    u_rows, i_rows = gather(
        user_inputs.reshape(B // IDX_CHUNK, IDX_CHUNK),
        item_inputs.reshape(B // IDX_CHUNK, IDX_CHUNK),
        user_table, item_table)

    bm = 4096
    y = pl.pallas_call(
        _mlp_body,
        grid=(B // bm,),
        in_specs=[
            pl.BlockSpec((bm, D), lambda i: (i, 0)),
            pl.BlockSpec((bm, DI), lambda i: (i, 0)),
            pl.BlockSpec((3 * D, 8), lambda i: (0, 0)),
            pl.BlockSpec((1, 8), lambda i: (0, 0)),
            pl.BlockSpec((8, 1), lambda i: (0, 0)),
            pl.BlockSpec((1, 1), lambda i: (0, 0)),
        ],
        out_specs=pl.BlockSpec((bm, 1), lambda i: (i, 0)),
        out_shape=jax.ShapeDtypeStruct((B, 1), jnp.float32),
    )(u_rows, i_rows, W1, b1.reshape(1, 8), W2, b2.reshape(1, 1))
    return y


# speedup vs baseline: 2.0866x; 2.0866x over previous
import functools

import jax
import jax.numpy as jnp
from jax import lax
from jax.experimental import pallas as pl
from jax.experimental.pallas import tpu as pltpu
from jax.experimental.pallas import tpu_sc as plsc

B = 16384
D = 32
DI = 14
NC = 2
NS = 16
NW = NC * NS
BPW = B // NW
CHUNK = BPW // 2


def _sc_gather(uidx_hbm, iidx_hbm, utab_hbm, itab_hbm, u_out, i_out,
               uidx_v, iidx_v, urows_v, irows_v, usem, isem):
    wid = lax.axis_index("s") * NC + lax.axis_index("c")
    base = wid * BPW
    pltpu.sync_copy(uidx_hbm.at[pl.ds(base, BPW)], uidx_v)
    pltpu.sync_copy(iidx_hbm.at[pl.ds(base, BPW)], iidx_v)

    for p in range(BPW // CHUNK):
        off = p * CHUNK

        def issue(g, _):
            uvec = uidx_v[pl.ds(off + g * 16, 16)]
            ivec = iidx_v[pl.ds(off + g * 16, 16)]
            for k in range(16):
                ui = uvec[k]
                pltpu.make_async_copy(
                    utab_hbm.at[pl.ds(ui, 1)],
                    urows_v.at[pl.ds(g * 16 + k, 1)], usem).start()
                ii = ivec[k]
                pltpu.make_async_copy(
                    itab_hbm.at[pl.ds(ii, 1)],
                    irows_v.at[pl.ds(g * 16 + k, 1)], isem).start()
            return ()

        lax.fori_loop(0, CHUNK // 16, issue, ())

        def drain(i, _):
            pltpu.make_async_copy(
                utab_hbm.at[pl.ds(0, 1)], urows_v.at[pl.ds(0, 1)],
                usem).wait()
            pltpu.make_async_copy(
                itab_hbm.at[pl.ds(0, 1)], irows_v.at[pl.ds(0, 1)],
                isem).wait()
            return ()

        lax.fori_loop(0, CHUNK, drain, (), unroll=4)
        pltpu.sync_copy(urows_v, u_out.at[pl.ds(base + off, CHUNK)])
        pltpu.sync_copy(irows_v, i_out.at[pl.ds(base + off, CHUNK)])


def _mlp_body(u_ref, it_ref, w1_ref, b1_ref, w2_ref, b2_ref, y_ref):
    u = u_ref[:]
    it = it_ref[:]
    w1 = w1_ref[:]
    elem = u[:, :DI] * it
    h = jnp.dot(elem, w1[0:DI, :], preferred_element_type=jnp.float32)
    h = h + jnp.dot(u, w1[D:2 * D, :], preferred_element_type=jnp.float32)
    h = h + jnp.dot(it, w1[2 * D:2 * D + DI, :],
                    preferred_element_type=jnp.float32)
    h = jnp.maximum(h + b1_ref[:], 0.0)
    z = jnp.dot(h, w2_ref[:], preferred_element_type=jnp.float32) + b2_ref[:]
    y_ref[:] = jax.nn.sigmoid(z)


def kernel(group_inputs, user_inputs, item_inputs, user_table, item_table,
           W1, b1, W2, b2):
    del group_inputs
    mesh = plsc.VectorSubcoreMesh(core_axis_name="c", subcore_axis_name="s")
    gather = functools.partial(
        pl.kernel,
        out_type=[
            jax.ShapeDtypeStruct((B, D), jnp.float32),
            jax.ShapeDtypeStruct((B, DI), jnp.float32),
        ],
        mesh=mesh,
        scratch_types=[
            pltpu.VMEM((BPW,), jnp.int32),
            pltpu.VMEM((BPW,), jnp.int32),
            pltpu.VMEM((CHUNK, D), jnp.float32),
            pltpu.VMEM((CHUNK, DI), jnp.float32),
            pltpu.SemaphoreType.DMA,
            pltpu.SemaphoreType.DMA,
        ],
    )(_sc_gather)
    u_rows, i_rows = gather(user_inputs, item_inputs, user_table, item_table)

    bm = 4096
    y = pl.pallas_call(
        _mlp_body,
        grid=(B // bm,),
        in_specs=[
            pl.BlockSpec((bm, D), lambda i: (i, 0)),
            pl.BlockSpec((bm, DI), lambda i: (i, 0)),
            pl.BlockSpec((3 * D, 8), lambda i: (0, 0)),
            pl.BlockSpec((1, 8), lambda i: (0, 0)),
            pl.BlockSpec((8, 1), lambda i: (0, 0)),
            pl.BlockSpec((1, 1), lambda i: (0, 0)),
        ],
        out_specs=pl.BlockSpec((bm, 1), lambda i: (i, 0)),
        out_shape=jax.ShapeDtypeStruct((B, 1), jnp.float32),
    )(u_rows, i_rows, W1, b1.reshape(1, 8), W2, b2.reshape(1, 1))
    return y
